# baseline (speedup 1.0000x reference)
import functools

import jax
import jax.numpy as jnp
from jax import lax
from jax.experimental import pallas as pl
from jax.experimental.pallas import tpu as pltpu
from jax.experimental.pallas import tpu_sc as plsc

N = 10000
E = 320000
DF = 128
DH = 16
NC = 4

LANES = 16
EPR = 80
ROWS = E // EPR
NW = 32
RPT = ROWS // NW
NPT = N // 16
NPAD = 640
NBUF = 5
MAIN = RPT // NBUF

_mesh = plsc.VectorSubcoreMesh(core_axis_name="c", subcore_axis_name="s")
_sc_params = pltpu.CompilerParams(needs_layout_passes=False,
                                  use_tc_tiling_on_sc=False)

_GDN = lax.GatherDimensionNumbers(
    offset_dims=(), collapsed_slice_dims=(0,), start_index_map=(0,))


def _splat(vec16, i):
    idx = jnp.full((LANES, 1), i, dtype=jnp.int32)
    return lax.gather(vec16, idx, _GDN, slice_sizes=(1,),
                      mode=lax.GatherScatterMode.PROMISE_IN_BOUNDS)


def _rsqrt_newton(d):
    i = plsc.bitcast(d, jnp.int32)
    i = jnp.int32(0x5F3759DF) - lax.shift_right_arithmetic(i, 1)
    y = plsc.bitcast(i, jnp.float32)
    for _ in range(3):
        y = y * (1.5 - 0.5 * d * y * y)
    return y


def _zero_rows(buf, nrows):
    def zero(i, _):
        buf[i, :] = jnp.zeros((LANES,), jnp.float32)
        return 0
    lax.fori_loop(0, nrows, zero, 0)


def _prop_pipeline(row_v, col_v, ew_v, src_sp, acc_sp, gbuf, sbuf, gsem, ssem):
    for t in range(NBUF):
        pltpu.async_copy(src_sp.at[row_v.at[t]], gbuf[t], gsem[t])

    def mbody(m, _):
        for t in range(NBUF):
            j = m * NBUF + t
            pltpu.make_async_copy(
                src_sp.at[row_v.at[j]], gbuf[t], gsem[t]).wait()

            @pl.when(m > 0)
            def _wait_scatter():
                pltpu.make_async_copy(
                    sbuf[t], acc_sp.at[col_v.at[j]], ssem[t]).wait()

            for k in range(EPR // LANES):
                w16 = ew_v[j, pl.ds(k * LANES, LANES)]
                for i in range(LANES):
                    e = k * LANES + i
                    sbuf[t][e, :] = gbuf[t][e, :] * _splat(w16, i)
            pltpu.async_copy(sbuf[t], acc_sp.at[col_v.at[j]], ssem[t],
                             add=True)

            @pl.when(m < MAIN - 1)
            def _next_gather():
                pltpu.async_copy(
                    src_sp.at[row_v.at[j + NBUF]], gbuf[t], gsem[t])
        return 0

    lax.fori_loop(0, MAIN, mbody, 0)
    for t in range(NBUF):
        jl = (MAIN - 1) * NBUF + t
        pltpu.make_async_copy(sbuf[t], acc_sp.at[col_v.at[jl]], ssem[t]).wait()


_PROP_SCRATCH = (
    [
        pltpu.VMEM((RPT, EPR), jnp.int32),
        pltpu.VMEM((RPT, EPR), jnp.int32),
        pltpu.VMEM((RPT, EPR), jnp.float32),
    ]
    + [pltpu.VMEM((EPR, DH), jnp.float32)] * (2 * NBUF)
    + [pltpu.SemaphoreType.DMA] * (2 * NBUF + 1)
)


@functools.partial(
    pl.kernel,
    mesh=_mesh,
    out_type=(jax.ShapeDtypeStruct((2, 16, NPT, DH), jnp.float32),
              jax.ShapeDtypeStruct((16, NPAD), jnp.float32),
              jax.ShapeDtypeStruct((2, N, DH), jnp.float32)),
    compiler_params=_sc_params,
    scratch_types=_PROP_SCRATCH + [
        pltpu.VMEM((RPT, EPR), jnp.int32),
        pltpu.VMEM((RPT, EPR), jnp.float32),
        pltpu.VMEM((N,), jnp.float32),
        pltpu.VMEM((NPAD + 16,), jnp.float32),
        pltpu.VMEM((NPAD,), jnp.float32),
        pltpu.VMEM((NPAD, DH), jnp.float32),
        pltpu.VMEM((NPAD, DH), jnp.float32),
        pltpu.VMEM_SHARED((16, N), jnp.float32),
        pltpu.VMEM_SHARED((N, DH), jnp.float32),
    ],
)
def _sc_layer1(xw_hbm, row_hbm, col_hbm, ew_hbm, acc_out, dinv_out, y_out,
               *scratch):
    row_v, col_v, ew_v = scratch[0:3]
    gbuf = scratch[3:3 + NBUF]
    sbuf = scratch[3 + NBUF:3 + 2 * NBUF]
    gsem = scratch[3 + 2 * NBUF:3 + 3 * NBUF]
    insem = scratch[3 + 4 * NBUF]
    ssem = scratch[3 + 3 * NBUF:3 + 4 * NBUF]
    (dcol_v, dew_v, hist_v, tmp_v, dinv_v, xbuf, ybuf,
     histx_sp, acc_sp) = scratch[4 + 4 * NBUF:]

    cid = lax.axis_index("c")
    sid = lax.axis_index("s")
    wid = cid * 16 + sid

    pltpu.async_copy(row_hbm.at[wid], row_v, insem)
    pltpu.async_copy(col_hbm.at[wid], col_v, insem)
    pltpu.async_copy(ew_hbm.at[wid], ew_v, insem)

    def hzero(g, _):
        hist_v[pl.ds(g * LANES, LANES)] = jnp.zeros((LANES,), jnp.float32)
        return 0

    lax.fori_loop(0, N // LANES, hzero, 0)

    for h in range(2):
        pltpu.sync_copy(col_hbm.at[2 * sid + h], dcol_v)
        pltpu.sync_copy(ew_hbm.at[2 * sid + h], dew_v)

        def hbody(j, _):
            for k in range(EPR // LANES):
                c16 = dcol_v[j, pl.ds(k * LANES, LANES)]
                w16 = dew_v[j, pl.ds(k * LANES, LANES)]
                plsc.addupdate_scatter(hist_v, [c16], w16)
            return 0

        lax.fori_loop(0, RPT, hbody, 0)

    pltpu.sync_copy(hist_v, histx_sp.at[sid])
    plsc.subcore_barrier()

    pad = sid % 8
    wstart = pl.multiple_of(sid * NPT - pad, 8)

    def dzero(g, _):
        dinv_v[pl.ds(g * LANES, LANES)] = jnp.zeros((LANES,), jnp.float32)
        return 0

    lax.fori_loop(0, NPAD // LANES, dzero, 0)
    for p in range(16):
        pltpu.sync_copy(histx_sp.at[p, pl.ds(wstart, 632)],
                        tmp_v.at[pl.ds(0, 632)])

        def abody(g, _):
            s = pl.ds(g * LANES, LANES)
            dinv_v[s] = dinv_v[s] + tmp_v[pl.ds(pad + g * LANES, LANES)]
            return 0

        lax.fori_loop(0, NPT // LANES + 1, abody, 0)

    def nbody(g, _):
        s = pl.ds(g * LANES, LANES)
        dinv_v[s] = _rsqrt_newton(dinv_v[s] + 1.0)
        return 0

    lax.fori_loop(0, NPAD // LANES, nbody, 0)

    @pl.when(cid == 0)
    def _write_dinv():
        pltpu.sync_copy(dinv_v, dinv_out.at[sid])

    pltpu.sync_copy(xw_hbm.at[pl.ds(sid * NPT, NPT)], xbuf.at[pl.ds(0, NPT)])

    def ybody(g, _):
        d16 = dinv_v[pl.ds(g * LANES, LANES)]
        for i in range(LANES):
            n = g * LANES + i
            ybuf[n, :] = xbuf[n, :] * _splat(d16, i)
        return 0

    lax.fori_loop(0, NPT // LANES + 1, ybody, 0)
    pltpu.sync_copy(ybuf.at[pl.ds(0, NPT)],
                    y_out.at[cid].at[pl.ds(sid * NPT, NPT)])

    _zero_rows(ybuf, NPT)
    pltpu.sync_copy(ybuf.at[pl.ds(0, NPT)], acc_sp.at[pl.ds(sid * NPT, NPT)])

    pltpu.make_async_copy(row_hbm.at[wid], row_v, insem).wait()
    pltpu.make_async_copy(col_hbm.at[wid], col_v, insem).wait()
    pltpu.make_async_copy(ew_hbm.at[wid], ew_v, insem).wait()
    plsc.subcore_barrier()

    _prop_pipeline(row_v, col_v, ew_v, y_out.at[cid], acc_sp,
                   gbuf, sbuf, gsem, ssem)
    plsc.subcore_barrier()
    pltpu.sync_copy(acc_sp.at[pl.ds(sid * NPT, NPT)], acc_out.at[cid, sid])


@functools.partial(
    pl.kernel,
    mesh=_mesh,
    out_type=(jax.ShapeDtypeStruct((2, 16, NPT, DH), jnp.float32),
              jax.ShapeDtypeStruct((2, N, DH), jnp.float32)),
    compiler_params=_sc_params,
    scratch_types=_PROP_SCRATCH + [
        pltpu.VMEM((NPAD, DH), jnp.float32),
        pltpu.VMEM((NPAD, DH), jnp.float32),
        pltpu.VMEM((NPAD, DH), jnp.float32),
        pltpu.VMEM((NPAD, DH), jnp.float32),
        pltpu.VMEM((NPAD,), jnp.float32),
        pltpu.VMEM((LANES,), jnp.float32),
        pltpu.VMEM_SHARED((N, DH), jnp.float32),
    ],
)
def _sc_layer2(xw_hbm, acc1_hbm, dinv_hbm, b1_hbm, row_hbm, col_hbm, ew_hbm,
               acc_out, g_out, *scratch):
    row_v, col_v, ew_v = scratch[0:3]
    gbuf = scratch[3:3 + NBUF]
    sbuf = scratch[3 + NBUF:3 + 2 * NBUF]
    gsem = scratch[3 + 2 * NBUF:3 + 3 * NBUF]
    ssem = scratch[3 + 3 * NBUF:3 + 4 * NBUF]
    insem = scratch[3 + 4 * NBUF]
    a0buf, a1buf, xbuf, gsbuf, dinv_v, b1_v, acc_sp = \
        scratch[4 + 4 * NBUF:]

    cid = lax.axis_index("c")
    sid = lax.axis_index("s")
    wid = cid * 16 + sid

    pltpu.async_copy(row_hbm.at[wid], row_v, insem)
    pltpu.async_copy(col_hbm.at[wid], col_v, insem)
    pltpu.async_copy(ew_hbm.at[wid], ew_v, insem)

    st = pl.ds(sid * NPT, NPT)
    pltpu.sync_copy(acc1_hbm.at[0, sid], a0buf.at[pl.ds(0, NPT)])
    pltpu.sync_copy(acc1_hbm.at[1, sid], a1buf.at[pl.ds(0, NPT)])
    pltpu.sync_copy(xw_hbm.at[st], xbuf.at[pl.ds(0, NPT)])
    pltpu.sync_copy(dinv_hbm.at[sid], dinv_v)
    pltpu.sync_copy(b1_hbm, b1_v)
    b1vec = b1_v[...]

    def hsbody(g, _):
        d16 = dinv_v[pl.ds(g * LANES, LANES)]
        for i in range(LANES):
            n = g * LANES + i
            d = _splat(d16, i)
            h = (a0buf[n, :] + a1buf[n, :] + d * xbuf[n, :]) * d + b1vec
            gsbuf[n, :] = d * jnp.maximum(h, 0.0)
        return 0

    lax.fori_loop(0, NPT // LANES + 1, hsbody, 0)
    pltpu.sync_copy(gsbuf.at[pl.ds(0, NPT)], g_out.at[cid].at[st])

    _zero_rows(gsbuf, NPT)
    pltpu.sync_copy(gsbuf.at[pl.ds(0, NPT)], acc_sp.at[st])

    pltpu.make_async_copy(row_hbm.at[wid], row_v, insem).wait()
    pltpu.make_async_copy(col_hbm.at[wid], col_v, insem).wait()
    pltpu.make_async_copy(ew_hbm.at[wid], ew_v, insem).wait()
    plsc.subcore_barrier()

    _prop_pipeline(row_v, col_v, ew_v, g_out.at[cid], acc_sp,
                   gbuf, sbuf, gsem, ssem)
    plsc.subcore_barrier()
    pltpu.sync_copy(acc_sp.at[pl.ds(sid * NPT, NPT)], acc_out.at[cid, sid])


def _tc_mm_body(x_ref, w1_ref, xw_ref):
    xw_ref[...] = jnp.dot(x_ref[...], w1_ref[...],
                          preferred_element_type=jnp.float32)


def _tc_final_body(c0_ref, c1_ref, g_ref, dinv_ref, b2_ref, w2_ref, out_ref):
    dinv = dinv_ref[...]
    t = c0_ref[...] + c1_ref[...] + g_ref[...]
    z = jnp.dot(t, w2_ref[...], preferred_element_type=jnp.float32) * dinv \
        + b2_ref[...]
    m = jnp.max(z, axis=1, keepdims=True)
    lse = jnp.log(jnp.sum(jnp.exp(z - m), axis=1, keepdims=True)) + m
    out_ref[...] = z - lse


def kernel(x, edge_index, edges_weight, W1, b1, W2, b2):
    row3d = edge_index[0].reshape(NW, RPT, EPR)
    col3d = edge_index[1].reshape(NW, RPT, EPR)
    ew3d = edges_weight.reshape(NW, RPT, EPR)

    xw1 = pl.pallas_call(
        _tc_mm_body,
        out_shape=jax.ShapeDtypeStruct((N, DH), jnp.float32),
    )(x, W1)

    acc1, dinv2d, _y1 = _sc_layer1(xw1, row3d, col3d, ew3d)

    acc2, g2 = _sc_layer2(xw1, acc1, dinv2d, b1, row3d, col3d, ew3d)

    a2r = acc2.reshape(2, N, DH)
    dinv_col = dinv2d[:, :NPT].reshape(N, 1)
    w2pad = jnp.zeros((DH, DH), jnp.float32).at[:, :NC].set(W2)
    b2pad = jnp.full((1, DH), -1e30, jnp.float32).at[0, :NC].set(b2)
    outp = pl.pallas_call(
        _tc_final_body,
        out_shape=jax.ShapeDtypeStruct((N, DH), jnp.float32),
    )(a2r[0], a2r[1], g2[0], dinv_col, b2pad, w2pad)

    return outp[:, :NC]


# speedup vs baseline: 1.1157x; 1.1157x over previous
import functools

import jax
import jax.numpy as jnp
from jax import lax
from jax.experimental import pallas as pl
from jax.experimental.pallas import tpu as pltpu
from jax.experimental.pallas import tpu_sc as plsc

N = 10000
E = 320000
DF = 128
DH = 16
NC = 4

LANES = 16
EPR = 80
ROWS = E // EPR
NW = 32
RPT = ROWS // NW
NPT = N // 16
NPAD = 640
NBUF = 5
MAIN = RPT // NBUF

_mesh = plsc.VectorSubcoreMesh(core_axis_name="c", subcore_axis_name="s")
_sc_params = pltpu.CompilerParams(needs_layout_passes=False,
                                  use_tc_tiling_on_sc=False)

_GDN = lax.GatherDimensionNumbers(
    offset_dims=(), collapsed_slice_dims=(0,), start_index_map=(0,))


def _splat(vec16, i):
    idx = jnp.full((LANES, 1), i, dtype=jnp.int32)
    return lax.gather(vec16, idx, _GDN, slice_sizes=(1,),
                      mode=lax.GatherScatterMode.PROMISE_IN_BOUNDS)


def _rsqrt_newton(d):
    i = plsc.bitcast(d, jnp.int32)
    i = jnp.int32(0x5F3759DF) - lax.shift_right_arithmetic(i, 1)
    y = plsc.bitcast(i, jnp.float32)
    for _ in range(3):
        y = y * (1.5 - 0.5 * d * y * y)
    return y


def _zero_rows(buf, nrows):
    def zero(i, _):
        buf[i, :] = jnp.zeros((LANES,), jnp.float32)
        return 0
    lax.fori_loop(0, nrows, zero, 0)


def _prop_pipeline(row_v, col_v, ew_v, src_sp, acc_sp, gbuf, sbuf, gsem, ssem):
    for t in range(NBUF):
        pltpu.async_copy(src_sp.at[row_v.at[t]], gbuf[t], gsem[t])

    def mbody(m, _):
        for t in range(NBUF):
            j = m * NBUF + t
            pltpu.make_async_copy(
                src_sp.at[row_v.at[j]], gbuf[t], gsem[t]).wait()

            @pl.when(m > 0)
            def _wait_scatter():
                pltpu.make_async_copy(
                    sbuf[t], acc_sp.at[col_v.at[j]], ssem[t]).wait()

            for k in range(EPR // LANES):
                w16 = ew_v[j, pl.ds(k * LANES, LANES)]
                for i in range(LANES):
                    e = k * LANES + i
                    sbuf[t][e, :] = gbuf[t][e, :] * _splat(w16, i)
            pltpu.async_copy(sbuf[t], acc_sp.at[col_v.at[j]], ssem[t],
                             add=True)

            @pl.when(m < MAIN - 1)
            def _next_gather():
                pltpu.async_copy(
                    src_sp.at[row_v.at[j + NBUF]], gbuf[t], gsem[t])
        return 0

    lax.fori_loop(0, MAIN, mbody, 0)
    for t in range(NBUF):
        jl = (MAIN - 1) * NBUF + t
        pltpu.make_async_copy(sbuf[t], acc_sp.at[col_v.at[jl]], ssem[t]).wait()


_PROP_SCRATCH = (
    [
        pltpu.VMEM((RPT, EPR), jnp.int32),
        pltpu.VMEM((RPT, EPR), jnp.int32),
        pltpu.VMEM((RPT, EPR), jnp.float32),
    ]
    + [pltpu.VMEM((EPR, DH), jnp.float32)] * (2 * NBUF)
    + [pltpu.SemaphoreType.DMA] * (2 * NBUF + 1)
)


@functools.partial(
    pl.kernel,
    mesh=_mesh,
    out_type=(jax.ShapeDtypeStruct((2, 16, NPT, DH), jnp.float32),
              jax.ShapeDtypeStruct((16, NPAD), jnp.float32)),
    compiler_params=_sc_params,
    scratch_types=_PROP_SCRATCH + [
        pltpu.VMEM((RPT, EPR), jnp.int32),
        pltpu.VMEM((RPT, EPR), jnp.float32),
        pltpu.VMEM((N,), jnp.float32),
        pltpu.VMEM((NPAD + 16,), jnp.float32),
        pltpu.VMEM((NPAD,), jnp.float32),
        pltpu.VMEM((NPAD, DH), jnp.float32),
        pltpu.VMEM((NPAD, DH), jnp.float32),
        pltpu.VMEM_SHARED((16, N), jnp.float32),
        pltpu.VMEM_SHARED((N, DH), jnp.float32),
        pltpu.VMEM_SHARED((N, DH), jnp.float32),
    ],
)
def _sc_layer1(xw_hbm, row_hbm, col_hbm, ew_hbm, acc_out, dinv_out,
               *scratch):
    row_v, col_v, ew_v = scratch[0:3]
    gbuf = scratch[3:3 + NBUF]
    sbuf = scratch[3 + NBUF:3 + 2 * NBUF]
    gsem = scratch[3 + 2 * NBUF:3 + 3 * NBUF]
    insem = scratch[3 + 4 * NBUF]
    ssem = scratch[3 + 3 * NBUF:3 + 4 * NBUF]
    (dcol_v, dew_v, hist_v, tmp_v, dinv_v, xbuf, ybuf,
     histx_sp, y_sp, acc_sp) = scratch[4 + 4 * NBUF:]

    cid = lax.axis_index("c")
    sid = lax.axis_index("s")
    wid = cid * 16 + sid

    pltpu.async_copy(row_hbm.at[wid], row_v, insem)
    pltpu.async_copy(col_hbm.at[wid], col_v, insem)
    pltpu.async_copy(ew_hbm.at[wid], ew_v, insem)

    def hzero(g, _):
        hist_v[pl.ds(g * LANES, LANES)] = jnp.zeros((LANES,), jnp.float32)
        return 0

    lax.fori_loop(0, N // LANES, hzero, 0)

    for h in range(2):
        pltpu.sync_copy(col_hbm.at[2 * sid + h], dcol_v)
        pltpu.sync_copy(ew_hbm.at[2 * sid + h], dew_v)

        def hbody(j, _):
            for k in range(EPR // LANES):
                c16 = dcol_v[j, pl.ds(k * LANES, LANES)]
                w16 = dew_v[j, pl.ds(k * LANES, LANES)]
                plsc.addupdate_scatter(hist_v, [c16], w16)
            return 0

        lax.fori_loop(0, RPT, hbody, 0)

    pltpu.sync_copy(hist_v, histx_sp.at[sid])
    plsc.subcore_barrier()

    pad = sid % 8
    wstart = pl.multiple_of(sid * NPT - pad, 8)

    def dzero(g, _):
        dinv_v[pl.ds(g * LANES, LANES)] = jnp.zeros((LANES,), jnp.float32)
        return 0

    lax.fori_loop(0, NPAD // LANES, dzero, 0)
    for p in range(16):
        pltpu.sync_copy(histx_sp.at[p, pl.ds(wstart, 632)],
                        tmp_v.at[pl.ds(0, 632)])

        def abody(g, _):
            s = pl.ds(g * LANES, LANES)
            dinv_v[s] = dinv_v[s] + tmp_v[pl.ds(pad + g * LANES, LANES)]
            return 0

        lax.fori_loop(0, NPT // LANES + 1, abody, 0)

    def nbody(g, _):
        s = pl.ds(g * LANES, LANES)
        dinv_v[s] = _rsqrt_newton(dinv_v[s] + 1.0)
        return 0

    lax.fori_loop(0, NPAD // LANES, nbody, 0)

    @pl.when(cid == 0)
    def _write_dinv():
        pltpu.sync_copy(dinv_v, dinv_out.at[sid])

    pltpu.sync_copy(xw_hbm.at[pl.ds(sid * NPT, NPT)], xbuf.at[pl.ds(0, NPT)])

    def ybody(g, _):
        d16 = dinv_v[pl.ds(g * LANES, LANES)]
        for i in range(LANES):
            n = g * LANES + i
            ybuf[n, :] = xbuf[n, :] * _splat(d16, i)
        return 0

    lax.fori_loop(0, NPT // LANES + 1, ybody, 0)
    pltpu.sync_copy(ybuf.at[pl.ds(0, NPT)], y_sp.at[pl.ds(sid * NPT, NPT)])

    _zero_rows(ybuf, NPT)
    pltpu.sync_copy(ybuf.at[pl.ds(0, NPT)], acc_sp.at[pl.ds(sid * NPT, NPT)])

    pltpu.make_async_copy(row_hbm.at[wid], row_v, insem).wait()
    pltpu.make_async_copy(col_hbm.at[wid], col_v, insem).wait()
    pltpu.make_async_copy(ew_hbm.at[wid], ew_v, insem).wait()
    plsc.subcore_barrier()

    _prop_pipeline(row_v, col_v, ew_v, y_sp, acc_sp, gbuf, sbuf, gsem, ssem)
    plsc.subcore_barrier()
    pltpu.sync_copy(acc_sp.at[pl.ds(sid * NPT, NPT)], acc_out.at[cid, sid])


@functools.partial(
    pl.kernel,
    mesh=_mesh,
    out_type=(jax.ShapeDtypeStruct((2, 16, NPT, DH), jnp.float32),
              jax.ShapeDtypeStruct((N, DH), jnp.float32)),
    compiler_params=_sc_params,
    scratch_types=_PROP_SCRATCH + [
        pltpu.VMEM((NPAD, DH), jnp.float32),
        pltpu.VMEM((NPAD, DH), jnp.float32),
        pltpu.VMEM((NPAD, DH), jnp.float32),
        pltpu.VMEM((NPAD, DH), jnp.float32),
        pltpu.VMEM((NPAD,), jnp.float32),
        pltpu.VMEM((LANES,), jnp.float32),
        pltpu.VMEM_SHARED((N, DH), jnp.float32),
        pltpu.VMEM_SHARED((N, DH), jnp.float32),
    ],
)
def _sc_layer2(xw_hbm, acc1_hbm, dinv_hbm, b1_hbm, row_hbm, col_hbm, ew_hbm,
               acc_out, g_out, *scratch):
    row_v, col_v, ew_v = scratch[0:3]
    gbuf = scratch[3:3 + NBUF]
    sbuf = scratch[3 + NBUF:3 + 2 * NBUF]
    gsem = scratch[3 + 2 * NBUF:3 + 3 * NBUF]
    ssem = scratch[3 + 3 * NBUF:3 + 4 * NBUF]
    insem = scratch[3 + 4 * NBUF]
    a0buf, a1buf, xbuf, gsbuf, dinv_v, b1_v, g_sp, acc_sp = \
        scratch[4 + 4 * NBUF:]

    cid = lax.axis_index("c")
    sid = lax.axis_index("s")
    wid = cid * 16 + sid

    pltpu.async_copy(row_hbm.at[wid], row_v, insem)
    pltpu.async_copy(col_hbm.at[wid], col_v, insem)
    pltpu.async_copy(ew_hbm.at[wid], ew_v, insem)

    st = pl.ds(sid * NPT, NPT)
    pltpu.sync_copy(acc1_hbm.at[0, sid], a0buf.at[pl.ds(0, NPT)])
    pltpu.sync_copy(acc1_hbm.at[1, sid], a1buf.at[pl.ds(0, NPT)])
    pltpu.sync_copy(xw_hbm.at[st], xbuf.at[pl.ds(0, NPT)])
    pltpu.sync_copy(dinv_hbm.at[sid], dinv_v)
    pltpu.sync_copy(b1_hbm, b1_v)
    b1vec = b1_v[...]

    def hsbody(g, _):
        d16 = dinv_v[pl.ds(g * LANES, LANES)]
        for i in range(LANES):
            n = g * LANES + i
            d = _splat(d16, i)
            h = (a0buf[n, :] + a1buf[n, :] + d * xbuf[n, :]) * d + b1vec
            gsbuf[n, :] = d * jnp.maximum(h, 0.0)
        return 0

    lax.fori_loop(0, NPT // LANES + 1, hsbody, 0)
    pltpu.sync_copy(gsbuf.at[pl.ds(0, NPT)], g_sp.at[st])

    @pl.when(cid == 0)
    def _write_g():
        pltpu.sync_copy(gsbuf.at[pl.ds(0, NPT)], g_out.at[st])

    _zero_rows(gsbuf, NPT)
    pltpu.sync_copy(gsbuf.at[pl.ds(0, NPT)], acc_sp.at[st])

    pltpu.make_async_copy(row_hbm.at[wid], row_v, insem).wait()
    pltpu.make_async_copy(col_hbm.at[wid], col_v, insem).wait()
    pltpu.make_async_copy(ew_hbm.at[wid], ew_v, insem).wait()
    plsc.subcore_barrier()

    _prop_pipeline(row_v, col_v, ew_v, g_sp, acc_sp, gbuf, sbuf, gsem, ssem)
    plsc.subcore_barrier()
    pltpu.sync_copy(acc_sp.at[pl.ds(sid * NPT, NPT)], acc_out.at[cid, sid])


def _tc_mm_body(x_ref, w1_ref, xw_ref):
    xw_ref[...] = jnp.dot(x_ref[...], w1_ref[...],
                          preferred_element_type=jnp.float32)


def _tc_final_body(c0_ref, c1_ref, g_ref, dinv_ref, b2_ref, w2_ref, out_ref):
    dinv = dinv_ref[...]
    t = c0_ref[...] + c1_ref[...] + g_ref[...]
    z = jnp.dot(t, w2_ref[...], preferred_element_type=jnp.float32) * dinv \
        + b2_ref[...]
    m = jnp.max(z, axis=1, keepdims=True)
    lse = jnp.log(jnp.sum(jnp.exp(z - m), axis=1, keepdims=True)) + m
    out_ref[...] = z - lse


def kernel(x, edge_index, edges_weight, W1, b1, W2, b2):
    row3d = edge_index[0].reshape(NW, RPT, EPR)
    col3d = edge_index[1].reshape(NW, RPT, EPR)
    ew3d = edges_weight.reshape(NW, RPT, EPR)

    xw1 = pl.pallas_call(
        _tc_mm_body,
        out_shape=jax.ShapeDtypeStruct((N, DH), jnp.float32),
    )(x, W1)

    acc1, dinv2d = _sc_layer1(xw1, row3d, col3d, ew3d)

    acc2, g2 = _sc_layer2(xw1, acc1, dinv2d, b1, row3d, col3d, ew3d)

    a2r = acc2.reshape(2, N, DH)
    dinv_col = dinv2d[:, :NPT].reshape(N, 1)
    w2pad = jnp.zeros((DH, DH), jnp.float32).at[:, :NC].set(W2)
    b2pad = jnp.full((1, DH), -1e30, jnp.float32).at[0, :NC].set(b2)
    outp = pl.pallas_call(
        _tc_final_body,
        out_shape=jax.ShapeDtypeStruct((N, DH), jnp.float32),
    )(a2r[0], a2r[1], g2, dinv_col, b2pad, w2pad)

    return outp[:, :NC]


# speedup vs baseline: 1.1223x; 1.0059x over previous
import functools

import jax
import jax.numpy as jnp
from jax import lax
from jax.experimental import pallas as pl
from jax.experimental.pallas import tpu as pltpu
from jax.experimental.pallas import tpu_sc as plsc

N = 10000
E = 320000
DF = 128
DH = 16
NC = 4

LANES = 16
EPR = 80
ROWS = E // EPR
NW = 32
RPT = ROWS // NW
NPT = N // 16
NPAD = 640
NBUF = 5
MAIN = RPT // NBUF

_mesh = plsc.VectorSubcoreMesh(core_axis_name="c", subcore_axis_name="s")
_sc_params = pltpu.CompilerParams(needs_layout_passes=False,
                                  use_tc_tiling_on_sc=False)

_GDN = lax.GatherDimensionNumbers(
    offset_dims=(), collapsed_slice_dims=(0,), start_index_map=(0,))


def _splat(vec16, i):
    idx = jnp.full((LANES, 1), i, dtype=jnp.int32)
    return lax.gather(vec16, idx, _GDN, slice_sizes=(1,),
                      mode=lax.GatherScatterMode.PROMISE_IN_BOUNDS)


def _rsqrt_newton(d):
    i = plsc.bitcast(d, jnp.int32)
    i = jnp.int32(0x5F3759DF) - lax.shift_right_arithmetic(i, 1)
    y = plsc.bitcast(i, jnp.float32)
    for _ in range(3):
        y = y * (1.5 - 0.5 * d * y * y)
    return y


def _zero_rows(buf, nrows):
    def zero(i, _):
        buf[i, :] = jnp.zeros((LANES,), jnp.float32)
        return 0
    lax.fori_loop(0, nrows, zero, 0)


def _prop_pipeline(row_v, col_v, ew_v, src_sp, acc_sp, gbuf, sbuf, gsem, ssem):
    for t in range(NBUF):
        pltpu.async_copy(src_sp.at[row_v.at[t]], gbuf[t], gsem[t])

    def mbody(m, _):
        for t in range(NBUF):
            j = m * NBUF + t
            pltpu.make_async_copy(
                src_sp.at[row_v.at[j]], gbuf[t], gsem[t]).wait()

            @pl.when(m > 0)
            def _wait_scatter():
                pltpu.make_async_copy(
                    sbuf[t], acc_sp.at[col_v.at[j]], ssem[t]).wait()

            for k in range(EPR // LANES):
                w16 = ew_v[j, pl.ds(k * LANES, LANES)]
                for i in range(LANES):
                    e = k * LANES + i
                    sbuf[t][e, :] = gbuf[t][e, :] * _splat(w16, i)
            pltpu.async_copy(sbuf[t], acc_sp.at[col_v.at[j]], ssem[t],
                             add=True)

            @pl.when(m < MAIN - 1)
            def _next_gather():
                pltpu.async_copy(
                    src_sp.at[row_v.at[j + NBUF]], gbuf[t], gsem[t])
        return 0

    lax.fori_loop(0, MAIN, mbody, 0)
    for t in range(NBUF):
        jl = (MAIN - 1) * NBUF + t
        pltpu.make_async_copy(sbuf[t], acc_sp.at[col_v.at[jl]], ssem[t]).wait()


_PROP_SCRATCH = (
    [
        pltpu.VMEM((RPT, EPR), jnp.int32),
        pltpu.VMEM((RPT, EPR), jnp.int32),
        pltpu.VMEM((RPT, EPR), jnp.float32),
    ]
    + [pltpu.VMEM((EPR, DH), jnp.float32)] * (2 * NBUF)
    + [pltpu.SemaphoreType.DMA] * (2 * NBUF + 1)
)


@functools.partial(
    pl.kernel,
    mesh=_mesh,
    out_type=(jax.ShapeDtypeStruct((2, 16, NPT, DH), jnp.float32),
              jax.ShapeDtypeStruct((16, NPAD), jnp.float32)),
    compiler_params=_sc_params,
    scratch_types=_PROP_SCRATCH + [
        pltpu.VMEM((RPT, EPR), jnp.int32),
        pltpu.VMEM((RPT, EPR), jnp.float32),
        pltpu.VMEM((N,), jnp.float32),
        pltpu.VMEM((NPAD + 16,), jnp.float32),
        pltpu.VMEM((NPAD,), jnp.float32),
        pltpu.VMEM((NPAD, DH), jnp.float32),
        pltpu.VMEM((NPAD, DH), jnp.float32),
        pltpu.VMEM_SHARED((16, N), jnp.float32),
        pltpu.VMEM_SHARED((N, DH), jnp.float32),
        pltpu.VMEM_SHARED((N, DH), jnp.float32),
    ],
)
def _sc_layer1(xw_hbm, row_hbm, col_hbm, ew_hbm, acc_out, dinv_out,
               *scratch):
    row_v, col_v, ew_v = scratch[0:3]
    gbuf = scratch[3:3 + NBUF]
    sbuf = scratch[3 + NBUF:3 + 2 * NBUF]
    gsem = scratch[3 + 2 * NBUF:3 + 3 * NBUF]
    insem = scratch[3 + 4 * NBUF]
    ssem = scratch[3 + 3 * NBUF:3 + 4 * NBUF]
    (dcol_v, dew_v, hist_v, tmp_v, dinv_v, xbuf, ybuf,
     histx_sp, y_sp, acc_sp) = scratch[4 + 4 * NBUF:]

    cid = lax.axis_index("c")
    sid = lax.axis_index("s")
    wid = cid * 16 + sid

    pltpu.async_copy(row_hbm.at[wid], row_v, insem)
    pltpu.async_copy(col_hbm.at[wid], col_v, insem)
    pltpu.async_copy(ew_hbm.at[wid], ew_v, insem)

    def hzero(g, _):
        hist_v[pl.ds(g * LANES, LANES)] = jnp.zeros((LANES,), jnp.float32)
        return 0

    lax.fori_loop(0, N // LANES, hzero, 0)

    for h in range(2):
        pltpu.sync_copy(col_hbm.at[2 * sid + h], dcol_v)
        pltpu.sync_copy(ew_hbm.at[2 * sid + h], dew_v)

        def hbody(j, _):
            for k in range(EPR // LANES):
                c16 = dcol_v[j, pl.ds(k * LANES, LANES)]
                w16 = dew_v[j, pl.ds(k * LANES, LANES)]
                plsc.addupdate_scatter(hist_v, [c16], w16)
            return 0

        lax.fori_loop(0, RPT, hbody, 0)

    pltpu.sync_copy(hist_v, histx_sp.at[sid])
    plsc.subcore_barrier()

    pad = sid % 8
    wstart = pl.multiple_of(sid * NPT - pad, 8)

    def dzero(g, _):
        dinv_v[pl.ds(g * LANES, LANES)] = jnp.zeros((LANES,), jnp.float32)
        return 0

    lax.fori_loop(0, NPAD // LANES, dzero, 0)
    for p in range(16):
        pltpu.sync_copy(histx_sp.at[p, pl.ds(wstart, 632)],
                        tmp_v.at[pl.ds(0, 632)])

        def abody(g, _):
            s = pl.ds(g * LANES, LANES)
            dinv_v[s] = dinv_v[s] + tmp_v[pl.ds(pad + g * LANES, LANES)]
            return 0

        lax.fori_loop(0, NPT // LANES + 1, abody, 0)

    def nbody(g, _):
        s = pl.ds(g * LANES, LANES)
        dinv_v[s] = _rsqrt_newton(dinv_v[s] + 1.0)
        return 0

    lax.fori_loop(0, NPAD // LANES, nbody, 0)

    @pl.when(cid == 0)
    def _write_dinv():
        pltpu.sync_copy(dinv_v, dinv_out.at[sid])

    pltpu.sync_copy(xw_hbm.at[pl.ds(sid * NPT, NPT)], xbuf.at[pl.ds(0, NPT)])

    def ybody(g, _):
        d16 = dinv_v[pl.ds(g * LANES, LANES)]
        for i in range(LANES):
            n = g * LANES + i
            ybuf[n, :] = xbuf[n, :] * _splat(d16, i)
        return 0

    lax.fori_loop(0, NPT // LANES + 1, ybody, 0)
    pltpu.sync_copy(ybuf.at[pl.ds(0, NPT)], y_sp.at[pl.ds(sid * NPT, NPT)])

    _zero_rows(ybuf, NPT)
    pltpu.sync_copy(ybuf.at[pl.ds(0, NPT)], acc_sp.at[pl.ds(sid * NPT, NPT)])

    pltpu.make_async_copy(row_hbm.at[wid], row_v, insem).wait()
    pltpu.make_async_copy(col_hbm.at[wid], col_v, insem).wait()
    pltpu.make_async_copy(ew_hbm.at[wid], ew_v, insem).wait()
    plsc.subcore_barrier()

    _prop_pipeline(row_v, col_v, ew_v, y_sp, acc_sp, gbuf, sbuf, gsem, ssem)
    plsc.subcore_barrier()
    pltpu.sync_copy(acc_sp.at[pl.ds(sid * NPT, NPT)], acc_out.at[cid, sid])


@functools.partial(
    pl.kernel,
    mesh=_mesh,
    out_type=(jax.ShapeDtypeStruct((2, 16, NPT, DH), jnp.float32),
              jax.ShapeDtypeStruct((N, DH), jnp.float32)),
    compiler_params=_sc_params,
    scratch_types=_PROP_SCRATCH + [
        pltpu.VMEM((NPAD, DH), jnp.float32),
        pltpu.VMEM((NPAD, DH), jnp.float32),
        pltpu.VMEM((NPAD, DH), jnp.float32),
        pltpu.VMEM((NPAD, DH), jnp.float32),
        pltpu.VMEM((NPAD,), jnp.float32),
        pltpu.VMEM((LANES,), jnp.float32),
        pltpu.VMEM_SHARED((N, DH), jnp.float32),
        pltpu.VMEM_SHARED((N, DH), jnp.float32),
    ],
)
def _sc_layer2(xw_hbm, acc1_hbm, dinv_hbm, b1_hbm, row_hbm, col_hbm, ew_hbm,
               acc_out, g_out, *scratch):
    row_v, col_v, ew_v = scratch[0:3]
    gbuf = scratch[3:3 + NBUF]
    sbuf = scratch[3 + NBUF:3 + 2 * NBUF]
    gsem = scratch[3 + 2 * NBUF:3 + 3 * NBUF]
    ssem = scratch[3 + 3 * NBUF:3 + 4 * NBUF]
    insem = scratch[3 + 4 * NBUF]
    a0buf, a1buf, xbuf, gsbuf, dinv_v, b1_v, g_sp, acc_sp = \
        scratch[4 + 4 * NBUF:]

    cid = lax.axis_index("c")
    sid = lax.axis_index("s")
    wid = cid * 16 + sid

    pltpu.async_copy(row_hbm.at[wid], row_v, insem)
    pltpu.async_copy(col_hbm.at[wid], col_v, insem)
    pltpu.async_copy(ew_hbm.at[wid], ew_v, insem)

    st = pl.ds(sid * NPT, NPT)
    pltpu.sync_copy(acc1_hbm.at[0, sid], a0buf.at[pl.ds(0, NPT)])
    pltpu.sync_copy(acc1_hbm.at[1, sid], a1buf.at[pl.ds(0, NPT)])
    pltpu.sync_copy(xw_hbm.at[st], xbuf.at[pl.ds(0, NPT)])
    pltpu.sync_copy(dinv_hbm.at[sid], dinv_v)
    pltpu.sync_copy(b1_hbm, b1_v)
    b1vec = b1_v[...]

    def hsbody(g, _):
        d16 = dinv_v[pl.ds(g * LANES, LANES)]
        for i in range(LANES):
            n = g * LANES + i
            d = _splat(d16, i)
            h = (a0buf[n, :] + a1buf[n, :] + d * xbuf[n, :]) * d + b1vec
            gsbuf[n, :] = d * jnp.maximum(h, 0.0)
        return 0

    lax.fori_loop(0, NPT // LANES + 1, hsbody, 0)
    pltpu.sync_copy(gsbuf.at[pl.ds(0, NPT)], g_sp.at[st])

    @pl.when(cid == 0)
    def _write_g():
        pltpu.sync_copy(gsbuf.at[pl.ds(0, NPT)], g_out.at[st])

    _zero_rows(gsbuf, NPT)
    pltpu.sync_copy(gsbuf.at[pl.ds(0, NPT)], acc_sp.at[st])

    pltpu.make_async_copy(row_hbm.at[wid], row_v, insem).wait()
    pltpu.make_async_copy(col_hbm.at[wid], col_v, insem).wait()
    pltpu.make_async_copy(ew_hbm.at[wid], ew_v, insem).wait()
    plsc.subcore_barrier()

    _prop_pipeline(row_v, col_v, ew_v, g_sp, acc_sp, gbuf, sbuf, gsem, ssem)
    plsc.subcore_barrier()
    pltpu.sync_copy(acc_sp.at[pl.ds(sid * NPT, NPT)], acc_out.at[cid, sid])


def _tc_mm_body(x_ref, w1_ref, xw_ref):
    xw_ref[...] = jnp.dot(x_ref[...], w1_ref[...],
                          preferred_element_type=jnp.float32)


def _tc_final_body(c0_ref, c1_ref, g_ref, dinv_ref, b2_ref, w2_ref, out_ref):
    dinv = dinv_ref[...]
    t = c0_ref[...] + c1_ref[...] + g_ref[...]
    z = jnp.dot(t, w2_ref[...], preferred_element_type=jnp.float32) * dinv \
        + b2_ref[...]
    m = jnp.max(z, axis=1, keepdims=True)
    lse = jnp.log(jnp.sum(jnp.exp(z - m), axis=1, keepdims=True)) + m
    out_ref[...] = z - lse


def _tc_final(acc2, g2, dinv_col, b2, W2):
    return pl.pallas_call(
        _tc_final_body,
        out_shape=jax.ShapeDtypeStruct((N, NC), jnp.float32),
    )(acc2[0], acc2[1], g2, dinv_col, b2.reshape(1, NC), W2)


def kernel(x, edge_index, edges_weight, W1, b1, W2, b2):
    row3d = edge_index[0].reshape(NW, RPT, EPR)
    col3d = edge_index[1].reshape(NW, RPT, EPR)
    ew3d = edges_weight.reshape(NW, RPT, EPR)

    xw1 = pl.pallas_call(
        _tc_mm_body,
        out_shape=jax.ShapeDtypeStruct((N, DH), jnp.float32),
    )(x, W1)

    acc1, dinv2d = _sc_layer1(xw1, row3d, col3d, ew3d)

    acc2, g2 = _sc_layer2(xw1, acc1, dinv2d, b1, row3d, col3d, ew3d)

    a2r = acc2.reshape(2, N, DH)
    dinv_col = dinv2d[:, :NPT].reshape(N, 1)
    return _tc_final(a2r, g2, dinv_col, b2, W2)
